# baseline (speedup 1.0000x reference)
import functools

import jax
import jax.numpy as jnp
from jax import lax
from jax.experimental import pallas as pl
from jax.experimental.pallas import tpu as pltpu
from jax.experimental.pallas import tpu_sc as plsc

BATCH_NUM = 1024
WIN_SIZE = 50
EMBED_DIM = 64
ROW_PAD = 128
N = 102400
NSEG = BATCH_NUM * WIN_SIZE

NC = 2
NS = 16
SUB = 128
CPT = N // SUB // NS

QTR = NSEG // 4
QTRASH = QTR
QZ = 804
QPAD = QZ * NS
QOUT = QTR // NS
QRW = 896

HALF = NSEG // NC
HTRASH = HALF
HZ = 1608
HPAD = HZ * NS
HOUT = HALF // NS
HRW = 1664
CW = 16


def _sums_body(ids_hbm, sq0, sq1, sq2, sq3, table_hbm, zeros_hbm,
               zramp_hbm, oramp_hbm, sums_out,
               ids_v, seg_v, rows_v, acc_sh, sem):
    c = lax.axis_index("c")
    s = lax.axis_index("s")

    for p in range(2):
        pltpu.sync_copy(zeros_hbm, rows_v)
        for i in range(QRW // SUB):
            pltpu.sync_copy(zramp_hbm.at[s, 0, pl.ds(i * SUB, SUB)], seg_v)
            pltpu.sync_copy(rows_v, acc_sh.at[seg_v])
        plsc.subcore_barrier()

        sq_a = sq0 if p == 0 else sq1
        sq_b = sq2 if p == 0 else sq3

        @pl.loop(0, CPT)
        def chunk_body(ch):
            base = (s * CPT + ch) * SUB
            pltpu.sync_copy(ids_hbm.at[pl.ds(base, SUB)], ids_v)

            @pl.when(c == 0)
            def _():
                pltpu.sync_copy(sq_a.at[pl.ds(base, SUB)], seg_v)

            @pl.when(c == 1)
            def _():
                pltpu.sync_copy(sq_b.at[pl.ds(base, SUB)], seg_v)

            pltpu.async_copy(table_hbm.at[ids_v], rows_v, sem).wait()
            pltpu.sync_copy(rows_v, acc_sh.at[seg_v], add=True)

        plsc.subcore_barrier()

        q = 2 * c + p
        out0 = q * QTR + s * QOUT
        for i in range(QRW // SUB):
            pltpu.sync_copy(oramp_hbm.at[s, 0, pl.ds(i * SUB, SUB)], seg_v)
            pltpu.async_copy(acc_sh.at[seg_v], rows_v, sem).wait()
            nrows = min(SUB, max(0, QOUT - i * SUB))
            if nrows == SUB:
                pltpu.sync_copy(rows_v,
                                sums_out.at[pl.ds(out0 + i * SUB, SUB)])
            elif nrows:
                pltpu.sync_copy(rows_v.at[pl.ds(0, nrows)],
                                sums_out.at[pl.ds(out0 + i * SUB, nrows)])
        if p == 0:
            plsc.subcore_barrier()


def _counts_body(sc0, sc1, ones_hbm, zeros_hbm, zramp_hbm, oramp_hbm,
                 counts_out, seg_v, ones_v, acc_sh, sem):
    c = lax.axis_index("c")
    s = lax.axis_index("s")

    pltpu.sync_copy(zeros_hbm, ones_v)
    for i in range(HRW // SUB):
        pltpu.sync_copy(zramp_hbm.at[s, 0, pl.ds(i * SUB, SUB)], seg_v)
        pltpu.sync_copy(ones_v, acc_sh.at[seg_v])
    plsc.subcore_barrier()

    pltpu.sync_copy(ones_hbm, ones_v)

    @pl.loop(0, CPT)
    def chunk_body(ch):
        base = (s * CPT + ch) * SUB

        @pl.when(c == 0)
        def _():
            pltpu.sync_copy(sc0.at[pl.ds(base, SUB)], seg_v)

        @pl.when(c == 1)
        def _():
            pltpu.sync_copy(sc1.at[pl.ds(base, SUB)], seg_v)

        pltpu.sync_copy(ones_v, acc_sh.at[seg_v], add=True)

    plsc.subcore_barrier()

    out0 = c * HALF + s * HOUT
    for i in range(HRW // SUB):
        pltpu.sync_copy(oramp_hbm.at[s, 0, pl.ds(i * SUB, SUB)], seg_v)
        pltpu.async_copy(acc_sh.at[seg_v], ones_v, sem).wait()
        nrows = min(SUB, max(0, HOUT - i * SUB))
        if nrows == SUB:
            pltpu.sync_copy(ones_v,
                            counts_out.at[pl.ds(out0 + i * SUB, SUB)])
        elif nrows:
            pltpu.sync_copy(ones_v.at[pl.ds(0, nrows)],
                            counts_out.at[pl.ds(out0 + i * SUB, nrows)])


_sc_sums = functools.partial(
    pl.kernel,
    mesh=plsc.VectorSubcoreMesh(core_axis_name="c", subcore_axis_name="s"),
    out_type=jax.ShapeDtypeStruct((NSEG, ROW_PAD), jnp.float32),
    scratch_types=[
        pltpu.VMEM((SUB,), jnp.int32),
        pltpu.VMEM((SUB,), jnp.int32),
        pltpu.VMEM((SUB, ROW_PAD), jnp.float32),
        pltpu.VMEM_SHARED((QPAD, ROW_PAD), jnp.float32),
        pltpu.SemaphoreType.DMA,
    ],
)(_sums_body)

_sc_counts = functools.partial(
    pl.kernel,
    mesh=plsc.VectorSubcoreMesh(core_axis_name="c", subcore_axis_name="s"),
    out_type=jax.ShapeDtypeStruct((NSEG, CW), jnp.float32),
    scratch_types=[
        pltpu.VMEM((SUB,), jnp.int32),
        pltpu.VMEM((SUB, CW), jnp.float32),
        pltpu.VMEM_SHARED((HPAD, CW), jnp.float32),
        pltpu.SemaphoreType.DMA,
    ],
)(_counts_body)


_BT = 16


def _mean_t_body(s_ref, o_ref):
    sm = s_ref[:, :, :EMBED_DIM]
    ct = s_ref[:, :, EMBED_DIM:EMBED_DIM + 1]
    m = sm / jnp.maximum(ct, 1.0)
    o_ref[...] = jnp.transpose(m, (0, 2, 1))


def kernel(input, batch_i, win_i, table):
    tab128 = jnp.pad(table, ((0, 0), (0, ROW_PAD - EMBED_DIM)))
    tab128 = tab128.at[:, EMBED_DIM].set(1.0)
    seg = batch_i * WIN_SIZE + win_i

    def local(lo):
        d = seg - lo
        return jnp.where((d >= 0) & (d < QTR), d, QTRASH).astype(jnp.int32)

    sq = [local(q * QTR) for q in range(4)]

    tiles = jnp.arange(NS, dtype=jnp.int32)[:, None, None]
    j_q = jnp.arange(QRW, dtype=jnp.int32)[None, None, :]
    zramp_q = jnp.where(j_q < QZ, tiles * QZ + j_q, QTRASH)
    oramp_q = jnp.where(j_q < QOUT, tiles * QOUT + j_q, QTRASH)

    zeros128 = jnp.zeros((SUB, ROW_PAD), jnp.float32)

    sums = _sc_sums(input, sq[0], sq[1], sq[2], sq[3], tab128, zeros128,
                    zramp_q, oramp_q)
    sums3 = sums.reshape(BATCH_NUM, WIN_SIZE, ROW_PAD)
    out = pl.pallas_call(
        _mean_t_body,
        grid=(BATCH_NUM // _BT,),
        in_specs=[
            pl.BlockSpec((_BT, WIN_SIZE, ROW_PAD), lambda i: (i, 0, 0)),
        ],
        out_specs=pl.BlockSpec((_BT, EMBED_DIM, WIN_SIZE), lambda i: (i, 0, 0)),
        out_shape=jax.ShapeDtypeStruct((BATCH_NUM, EMBED_DIM, WIN_SIZE),
                                       jnp.float32),
    )(sums3)
    return out


# speedup vs baseline: 1.0026x; 1.0026x over previous
import functools

import jax
import jax.numpy as jnp
from jax import lax
from jax.experimental import pallas as pl
from jax.experimental.pallas import tpu as pltpu
from jax.experimental.pallas import tpu_sc as plsc

BATCH_NUM = 1024
WIN_SIZE = 50
EMBED_DIM = 64
ROW_PAD = 128
N = 102400
NSEG = BATCH_NUM * WIN_SIZE

NC = 2
NS = 16
SUB = 128
CPT = N // SUB // NS

QTR = NSEG // 4
QTRASH = QTR
QZ = 804
QPAD = QZ * NS
QOUT = QTR // NS
QRW = 896

HALF = NSEG // NC
HTRASH = HALF
HZ = 1608
HPAD = HZ * NS
HOUT = HALF // NS
HRW = 1664
CW = 16


def _sums_body(ids_hbm, sq0, sq1, sq2, sq3, table_hbm, zeros_hbm,
               zramp_hbm, oramp_hbm, sums_out,
               ids_v, seg_v, rows_v, acc_sh, sem):
    c = lax.axis_index("c")
    s = lax.axis_index("s")

    for p in range(2):
        pltpu.sync_copy(zeros_hbm, rows_v)
        for i in range(QRW // SUB):
            pltpu.sync_copy(zramp_hbm.at[s, 0, pl.ds(i * SUB, SUB)], seg_v)
            pltpu.sync_copy(rows_v, acc_sh.at[seg_v])
        plsc.subcore_barrier()

        sq_a = sq0 if p == 0 else sq1
        sq_b = sq2 if p == 0 else sq3

        @pl.loop(0, CPT)
        def chunk_body(ch):
            base = (s * CPT + ch) * SUB
            pltpu.sync_copy(ids_hbm.at[pl.ds(base, SUB)], ids_v)

            @pl.when(c == 0)
            def _():
                pltpu.sync_copy(sq_a.at[pl.ds(base, SUB)], seg_v)

            @pl.when(c == 1)
            def _():
                pltpu.sync_copy(sq_b.at[pl.ds(base, SUB)], seg_v)

            pltpu.async_copy(table_hbm.at[ids_v], rows_v, sem).wait()
            pltpu.sync_copy(rows_v, acc_sh.at[seg_v], add=True)

        plsc.subcore_barrier()

        q = 2 * c + p
        out0 = q * QTR + s * QOUT
        for i in range(QRW // SUB):
            pltpu.sync_copy(oramp_hbm.at[s, 0, pl.ds(i * SUB, SUB)], seg_v)
            pltpu.async_copy(acc_sh.at[seg_v], rows_v, sem).wait()
            nrows = min(SUB, max(0, QOUT - i * SUB))
            if nrows == SUB:
                pltpu.sync_copy(rows_v,
                                sums_out.at[pl.ds(out0 + i * SUB, SUB)])
            elif nrows:
                pltpu.sync_copy(rows_v.at[pl.ds(0, nrows)],
                                sums_out.at[pl.ds(out0 + i * SUB, nrows)])
        if p == 0:
            plsc.subcore_barrier()


def _counts_body(sc0, sc1, ones_hbm, zeros_hbm, zramp_hbm, oramp_hbm,
                 counts_out, seg_v, ones_v, acc_sh, sem):
    c = lax.axis_index("c")
    s = lax.axis_index("s")

    pltpu.sync_copy(zeros_hbm, ones_v)
    for i in range(HRW // SUB):
        pltpu.sync_copy(zramp_hbm.at[s, 0, pl.ds(i * SUB, SUB)], seg_v)
        pltpu.sync_copy(ones_v, acc_sh.at[seg_v])
    plsc.subcore_barrier()

    pltpu.sync_copy(ones_hbm, ones_v)

    @pl.loop(0, CPT)
    def chunk_body(ch):
        base = (s * CPT + ch) * SUB

        @pl.when(c == 0)
        def _():
            pltpu.sync_copy(sc0.at[pl.ds(base, SUB)], seg_v)

        @pl.when(c == 1)
        def _():
            pltpu.sync_copy(sc1.at[pl.ds(base, SUB)], seg_v)

        pltpu.sync_copy(ones_v, acc_sh.at[seg_v], add=True)

    plsc.subcore_barrier()

    out0 = c * HALF + s * HOUT
    for i in range(HRW // SUB):
        pltpu.sync_copy(oramp_hbm.at[s, 0, pl.ds(i * SUB, SUB)], seg_v)
        pltpu.async_copy(acc_sh.at[seg_v], ones_v, sem).wait()
        nrows = min(SUB, max(0, HOUT - i * SUB))
        if nrows == SUB:
            pltpu.sync_copy(ones_v,
                            counts_out.at[pl.ds(out0 + i * SUB, SUB)])
        elif nrows:
            pltpu.sync_copy(ones_v.at[pl.ds(0, nrows)],
                            counts_out.at[pl.ds(out0 + i * SUB, nrows)])


_sc_sums = functools.partial(
    pl.kernel,
    mesh=plsc.VectorSubcoreMesh(core_axis_name="c", subcore_axis_name="s"),
    out_type=jax.ShapeDtypeStruct((NSEG, ROW_PAD), jnp.float32),
    scratch_types=[
        pltpu.VMEM((SUB,), jnp.int32),
        pltpu.VMEM((SUB,), jnp.int32),
        pltpu.VMEM((SUB, ROW_PAD), jnp.float32),
        pltpu.VMEM_SHARED((QPAD, ROW_PAD), jnp.float32),
        pltpu.SemaphoreType.DMA,
    ],
)(_sums_body)

_sc_counts = functools.partial(
    pl.kernel,
    mesh=plsc.VectorSubcoreMesh(core_axis_name="c", subcore_axis_name="s"),
    out_type=jax.ShapeDtypeStruct((NSEG, CW), jnp.float32),
    scratch_types=[
        pltpu.VMEM((SUB,), jnp.int32),
        pltpu.VMEM((SUB, CW), jnp.float32),
        pltpu.VMEM_SHARED((HPAD, CW), jnp.float32),
        pltpu.SemaphoreType.DMA,
    ],
)(_counts_body)


_BT = 16


def _mean_t_body(s_ref, o_ref):
    sm = s_ref[:, :, :EMBED_DIM]
    ct = s_ref[:, :, EMBED_DIM:EMBED_DIM + 1]
    m = sm / jnp.maximum(ct, 1.0)
    o_ref[...] = jnp.transpose(m, (0, 2, 1))


def kernel(input, batch_i, win_i, table):
    tab128 = jnp.pad(table, ((0, 0), (0, ROW_PAD - EMBED_DIM)))
    tab128 = tab128.at[:, EMBED_DIM].set(1.0)
    seg = batch_i * WIN_SIZE + win_i

    spread = QTRASH + (jnp.arange(N, dtype=jnp.int32) & 63)

    def local(lo):
        d = seg - lo
        return jnp.where((d >= 0) & (d < QTR), d, spread).astype(jnp.int32)

    sq = [local(q * QTR) for q in range(4)]

    tiles = jnp.arange(NS, dtype=jnp.int32)[:, None, None]
    j_q = jnp.arange(QRW, dtype=jnp.int32)[None, None, :]
    zramp_q = jnp.where(j_q < QZ, tiles * QZ + j_q, QTRASH)
    oramp_q = jnp.where(j_q < QOUT, tiles * QOUT + j_q, QTRASH)

    zeros128 = jnp.zeros((SUB, ROW_PAD), jnp.float32)

    sums = _sc_sums(input, sq[0], sq[1], sq[2], sq[3], tab128, zeros128,
                    zramp_q, oramp_q)
    sums3 = sums.reshape(BATCH_NUM, WIN_SIZE, ROW_PAD)
    out = pl.pallas_call(
        _mean_t_body,
        grid=(BATCH_NUM // _BT,),
        in_specs=[
            pl.BlockSpec((_BT, WIN_SIZE, ROW_PAD), lambda i: (i, 0, 0)),
        ],
        out_specs=pl.BlockSpec((_BT, EMBED_DIM, WIN_SIZE), lambda i: (i, 0, 0)),
        out_shape=jax.ShapeDtypeStruct((BATCH_NUM, EMBED_DIM, WIN_SIZE),
                                       jnp.float32),
    )(sums3)
    return out


# speedup vs baseline: 2.6384x; 2.6315x over previous
import functools

import jax
import jax.numpy as jnp
from jax import lax
from jax.experimental import pallas as pl
from jax.experimental.pallas import tpu as pltpu
from jax.experimental.pallas import tpu_sc as plsc

BATCH_NUM = 1024
WIN_SIZE = 50
EMBED_DIM = 64
ROW_PAD = 128
N = 102400
NSEG = BATCH_NUM * WIN_SIZE

NC = 2
NS = 16
SUB = 128
CPT = N // SUB // NS

QTR = NSEG // 4
QTRASH = QTR
QZ = 804
QPAD = QZ * NS
QOUT = QTR // NS
QRW = 896

HALF = NSEG // NC
HTRASH = HALF
HZ = 1608
HPAD = HZ * NS
HOUT = HALF // NS
HRW = 1664
CW = 16


def _sums_body(ids_hbm, sq0, sq1, sq2, sq3, table_hbm, zeros_hbm,
               zramp_hbm, oramp_hbm, sums_out,
               ids_v, seg_v, rows_v, acc_sh, sem):
    c = lax.axis_index("c")
    s = lax.axis_index("s")

    for p in range(2):
        pltpu.sync_copy(zeros_hbm, rows_v)
        for i in range(QRW // SUB):
            pltpu.sync_copy(zramp_hbm.at[s, 0, pl.ds(i * SUB, SUB)], seg_v)
            pltpu.sync_copy(rows_v, acc_sh.at[seg_v])
        plsc.subcore_barrier()

        sq_a = sq0 if p == 0 else sq1
        sq_b = sq2 if p == 0 else sq3

        @pl.loop(0, CPT)
        def chunk_body(ch):
            base = (s * CPT + ch) * SUB
            pltpu.sync_copy(ids_hbm.at[pl.ds(base, SUB)], ids_v)

            @pl.when(c == 0)
            def _():
                pltpu.sync_copy(sq_a.at[pl.ds(base, SUB)], seg_v)

            @pl.when(c == 1)
            def _():
                pltpu.sync_copy(sq_b.at[pl.ds(base, SUB)], seg_v)

            pltpu.async_copy(table_hbm.at[ids_v], rows_v, sem).wait()
            pltpu.sync_copy(rows_v, acc_sh.at[seg_v], add=True)

        plsc.subcore_barrier()

        q = 2 * c + p
        out0 = q * QTR + s * QOUT
        for i in range(QRW // SUB):
            pltpu.sync_copy(oramp_hbm.at[s, 0, pl.ds(i * SUB, SUB)], seg_v)
            pltpu.async_copy(acc_sh.at[seg_v], rows_v, sem).wait()
            nrows = min(SUB, max(0, QOUT - i * SUB))
            if nrows == SUB:
                pltpu.sync_copy(rows_v,
                                sums_out.at[pl.ds(out0 + i * SUB, SUB)])
            elif nrows:
                pltpu.sync_copy(rows_v.at[pl.ds(0, nrows)],
                                sums_out.at[pl.ds(out0 + i * SUB, nrows)])
        if p == 0:
            plsc.subcore_barrier()


def _counts_body(sc0, sc1, ones_hbm, zeros_hbm, zramp_hbm, oramp_hbm,
                 counts_out, seg_v, ones_v, acc_sh, sem):
    c = lax.axis_index("c")
    s = lax.axis_index("s")

    pltpu.sync_copy(zeros_hbm, ones_v)
    for i in range(HRW // SUB):
        pltpu.sync_copy(zramp_hbm.at[s, 0, pl.ds(i * SUB, SUB)], seg_v)
        pltpu.sync_copy(ones_v, acc_sh.at[seg_v])
    plsc.subcore_barrier()

    pltpu.sync_copy(ones_hbm, ones_v)

    @pl.loop(0, CPT)
    def chunk_body(ch):
        base = (s * CPT + ch) * SUB

        @pl.when(c == 0)
        def _():
            pltpu.sync_copy(sc0.at[pl.ds(base, SUB)], seg_v)

        @pl.when(c == 1)
        def _():
            pltpu.sync_copy(sc1.at[pl.ds(base, SUB)], seg_v)

        pltpu.sync_copy(ones_v, acc_sh.at[seg_v], add=True)

    plsc.subcore_barrier()

    out0 = c * HALF + s * HOUT
    for i in range(HRW // SUB):
        pltpu.sync_copy(oramp_hbm.at[s, 0, pl.ds(i * SUB, SUB)], seg_v)
        pltpu.async_copy(acc_sh.at[seg_v], ones_v, sem).wait()
        nrows = min(SUB, max(0, HOUT - i * SUB))
        if nrows == SUB:
            pltpu.sync_copy(ones_v,
                            counts_out.at[pl.ds(out0 + i * SUB, SUB)])
        elif nrows:
            pltpu.sync_copy(ones_v.at[pl.ds(0, nrows)],
                            counts_out.at[pl.ds(out0 + i * SUB, nrows)])


_sc_sums = functools.partial(
    pl.kernel,
    mesh=plsc.VectorSubcoreMesh(core_axis_name="c", subcore_axis_name="s"),
    out_type=jax.ShapeDtypeStruct((NSEG, ROW_PAD), jnp.float32),
    scratch_types=[
        pltpu.VMEM((SUB,), jnp.int32),
        pltpu.VMEM((SUB,), jnp.int32),
        pltpu.VMEM((SUB, ROW_PAD), jnp.float32),
        pltpu.VMEM_SHARED((QPAD, ROW_PAD), jnp.float32),
        pltpu.SemaphoreType.DMA,
    ],
)(_sums_body)

_sc_counts = functools.partial(
    pl.kernel,
    mesh=plsc.VectorSubcoreMesh(core_axis_name="c", subcore_axis_name="s"),
    out_type=jax.ShapeDtypeStruct((NSEG, CW), jnp.float32),
    scratch_types=[
        pltpu.VMEM((SUB,), jnp.int32),
        pltpu.VMEM((SUB, CW), jnp.float32),
        pltpu.VMEM_SHARED((HPAD, CW), jnp.float32),
        pltpu.SemaphoreType.DMA,
    ],
)(_counts_body)


_BT = 16


def _mean_t_body(s_ref, o_ref):
    sm = s_ref[:, :, :EMBED_DIM]
    ct = s_ref[:, :, EMBED_DIM:EMBED_DIM + 1]
    m = sm / jnp.maximum(ct, 1.0)
    o_ref[...] = jnp.transpose(m, (0, 2, 1))


def kernel(input, batch_i, win_i, table):
    vocab = table.shape[0]
    tab128 = jnp.concatenate(
        [table,
         jnp.ones((vocab, 1), jnp.float32),
         jnp.zeros((vocab, ROW_PAD - EMBED_DIM - 1), jnp.float32)], axis=1)
    seg = batch_i * WIN_SIZE + win_i

    spread = QTRASH + (jnp.arange(N, dtype=jnp.int32) & 63)

    def local(lo):
        d = seg - lo
        return jnp.where((d >= 0) & (d < QTR), d, spread).astype(jnp.int32)

    sq = [local(q * QTR) for q in range(4)]

    tiles = jnp.arange(NS, dtype=jnp.int32)[:, None, None]
    j_q = jnp.arange(QRW, dtype=jnp.int32)[None, None, :]
    zramp_q = jnp.where(j_q < QZ, tiles * QZ + j_q, QTRASH)
    oramp_q = jnp.where(j_q < QOUT, tiles * QOUT + j_q, QTRASH)

    zeros128 = jnp.zeros((SUB, ROW_PAD), jnp.float32)

    sums = _sc_sums(input, sq[0], sq[1], sq[2], sq[3], tab128, zeros128,
                    zramp_q, oramp_q)
    sums3 = sums.reshape(BATCH_NUM, WIN_SIZE, ROW_PAD)
    out = pl.pallas_call(
        _mean_t_body,
        grid=(BATCH_NUM // _BT,),
        in_specs=[
            pl.BlockSpec((_BT, WIN_SIZE, ROW_PAD), lambda i: (i, 0, 0)),
        ],
        out_specs=pl.BlockSpec((_BT, EMBED_DIM, WIN_SIZE), lambda i: (i, 0, 0)),
        out_shape=jax.ShapeDtypeStruct((BATCH_NUM, EMBED_DIM, WIN_SIZE),
                                       jnp.float32),
    )(sums3)
    return out
